# Initial kernel scaffold; baseline (speedup 1.0000x reference)
import jax
import jax.numpy as jnp
from jax.experimental import pallas as pl


def kernel(src, tgt):
    raise NotImplementedError("write your pallas kernel here")


# speedup vs baseline: 53.9254x; 53.9254x over previous
import functools

import jax
import jax.numpy as jnp
import numpy as np
from jax import lax
from jax.experimental import pallas as pl
from jax.experimental.pallas import tpu as pltpu
from jax.experimental.pallas import tpu_sc as plsc

NBINS = 256
NC, NS = 2, 16
NW = NC * NS
L = 16

_FC_NP = np.clip(np.arange(-1.0, 1.01, 1.0 / 127.0), -1.0, 1.0).astype(np.float32)

_HSLOTS = 2 * 3 * NBINS


def _worker_id():
    return lax.axis_index("s") * NC + lax.axis_index("c")


def _bin_index(x):
    d = (x + jnp.float32(1.0)) / jnp.float32(2.0) * jnp.float32(256.0)
    d = jnp.minimum(jnp.maximum(d, jnp.float32(0.0)), jnp.float32(255.0))
    return d.astype(jnp.int32)


def _hist_body(n_per_w, src_hbm, tgt_hbm, hp_hbm, sv, tv, hb, hc, sem):
    wid = _worker_id()
    base = wid * n_per_w
    c1 = pltpu.async_copy(src_hbm.at[pl.ds(base, n_per_w)], sv, sem)
    c2 = pltpu.async_copy(tgt_hbm.at[pl.ds(base, n_per_w)], tv, sem)
    c1.wait()
    c2.wait()

    lanes = lax.iota(jnp.int32, L)
    ones = jnp.full((L,), 1.0, jnp.float32)
    zeros = jnp.zeros((L,), jnp.float32)

    def zero_body(i, _):
        hb[pl.ds(i * L, L)] = zeros
        return 0

    lax.fori_loop(0, L * _HSLOTS // L, zero_body, 0)

    nvec = n_per_w // L

    def px_body(i, _):
        off = i * L
        ch = lax.rem(base + off + lanes, 3)
        lane_slot = lanes * _HSLOTS + ch * NBINS
        x = sv[pl.ds(off, L)]
        plsc.addupdate_scatter(hb, [lane_slot + _bin_index(x)], ones)
        y = tv[pl.ds(off, L)]
        plsc.addupdate_scatter(hb, [lane_slot + (3 * NBINS) + _bin_index(y)], ones)
        return 0

    lax.fori_loop(0, nvec, px_body, 0)

    def red_body(s, _):
        acc = hb[pl.ds(s * L, L)]
        for l in range(1, L):
            acc = acc + hb[pl.ds(l * _HSLOTS + s * L, L)]
        hc[pl.ds(s * L, L)] = acc
        return 0

    lax.fori_loop(0, _HSLOTS // L, red_body, 0)

    pltpu.sync_copy(hc, hp_hbm.at[pl.ds(wid * _HSLOTS, _HSLOTS)])


def _lower_bound(a_ref, a_off, x):
    lb = jnp.zeros((L,), jnp.int32)
    step = NBINS
    while step >= 1:
        probe = lb + (step - 1)
        inb = probe < NBINS
        pv = plsc.load_gather(a_ref, [a_off + jnp.minimum(probe, NBINS - 1)])
        take = jnp.logical_and(inb, pv < x)
        lb = jnp.where(take, lb + step, lb)
        step //= 2
    return lb


def _apply_body(n_per_w, npix_c, src_hbm, hp_hbm, fc_hbm, out_hbm,
                sv, ov, hg, gh, cdfv, pxv, fcv, sem):
    wid = _worker_id()
    base = wid * n_per_w
    c1 = pltpu.async_copy(src_hbm.at[pl.ds(base, n_per_w)], sv, sem)
    c2 = pltpu.async_copy(hp_hbm, hg, sem)
    c3 = pltpu.async_copy(fc_hbm, fcv, sem)
    c1.wait()
    c2.wait()
    c3.wait()

    lanes = lax.iota(jnp.int32, L)
    f0 = jnp.float32(0.0)
    scale = jnp.float32(2.0)
    denom = jnp.float32(npix_c - 1)

    def red_body(s, _):
        acc = hg[pl.ds(s * L, L)]
        for w in range(1, NW):
            acc = acc + hg[pl.ds(w * _HSLOTS + s * L, L)]
        gh[pl.ds(s * L, L)] = acc
        return 0

    lax.fori_loop(0, _HSLOTS // L, red_body, 0)

    for slot in range(6):
        off = slot * NBINS
        carry = f0
        cdf0 = f0
        for k in range(NBINS // L):
            v = gh[pl.ds(off + k * L, L)]
            cs = plsc.cumsum(v) + carry
            if k == 0:
                cdf0 = jnp.sum(jnp.where(lanes == 0, cs, f0))
            carry = carry + jnp.sum(v)
            cdfv[pl.ds(off + k * L, L)] = (cs - cdf0) * scale / denom - jnp.float32(1.0)

    for ch in range(3):
        a_off = (3 + ch) * NBINS

        def q_body(q, _, a_off=a_off, ch=ch):
            x = cdfv[pl.ds(ch * NBINS + q * L, L)]
            lb = _lower_bound(cdfv, a_off, x)
            lbc = jnp.minimum(jnp.maximum(lb, 1), NBINS - 1)
            v1 = plsc.load_gather(cdfv, [a_off + lbc - 1])
            v2 = plsc.load_gather(cdfv, [a_off + lbc])
            below = jnp.abs(v1 - x) <= jnp.abs(v2 - x)
            fo1 = _lower_bound(cdfv, a_off, v1)
            ind1 = jnp.where(below, fo1, lbc)
            ind1 = jnp.where(lb == 0, 0, ind1)
            ind0 = jnp.maximum(ind1 - 1, 0)
            a0 = plsc.load_gather(cdfv, [a_off + ind0])
            a1 = plsc.load_gather(cdfv, [a_off + ind1])
            y0 = plsc.load_gather(fcv, [ind0])
            y1 = plsc.load_gather(fcv, [ind1])
            inner = y0 + (y1 - y0) * (x - a0) / (a1 - a0)
            atop = plsc.load_gather(cdfv, [jnp.full((L,), a_off + NBINS - 1, jnp.int32)])
            res = jnp.where(x <= jnp.float32(-1.0), jnp.float32(-1.0),
                            jnp.where(x >= atop, jnp.float32(1.0), inner))
            pxv[pl.ds(ch * NBINS + q * L, L)] = res
            return 0

        lax.fori_loop(0, NBINS // L, q_body, 0)

    inf = jnp.float32(np.inf)

    def px_body(i, _):
        off = i * L
        x = sv[pl.ds(off, L)]
        ch = lax.rem(base + off + lanes, 3)
        choff = ch * NBINS
        t = (x + jnp.float32(1.0)) * jnp.float32(127.0)
        t = jnp.minimum(jnp.maximum(t, f0), jnp.float32(255.0))
        k0 = jnp.minimum(jnp.maximum(t.astype(jnp.int32) - 1, 0), NBINS - 4)
        bd = jnp.full((L,), inf)
        bi = jnp.zeros((L,), jnp.int32)
        fc1 = jnp.zeros((L,), jnp.float32)
        for o in range(4):
            ki = k0 + o
            fck = plsc.load_gather(fcv, [ki])
            d = jnp.abs(fck - x)
            upd = d < bd
            bd = jnp.where(upd, d, bd)
            bi = jnp.where(upd, ki, bi)
            fc1 = jnp.where(upd, fck, fc1)
        ind0 = jnp.maximum(bi - 1, 0)
        fc0 = plsc.load_gather(fcv, [ind0])
        p0 = plsc.load_gather(pxv, [choff + ind0])
        p1 = plsc.load_gather(pxv, [choff + bi])
        inner = p0 + (p1 - p0) * (x - fc0) / (fc1 - fc0)
        plo = plsc.load_gather(pxv, [choff])
        phi = plsc.load_gather(pxv, [choff + NBINS - 1])
        res = jnp.where(x <= jnp.float32(-1.0), plo,
                        jnp.where(x >= jnp.float32(1.0), phi, inner))
        ov[pl.ds(off, L)] = res
        return 0

    lax.fori_loop(0, n_per_w // L, px_body, 0)

    pltpu.sync_copy(ov, out_hbm.at[pl.ds(base, n_per_w)])


@jax.jit
def _run(srcf, tgtf, fc):
    npix = srcf.shape[0]
    npix_c = npix // 3
    n_per_w = npix // NW
    mesh = plsc.VectorSubcoreMesh(
        core_axis_name="c", subcore_axis_name="s", num_cores=NC, num_subcores=NS)
    cparams = pltpu.CompilerParams(needs_layout_passes=False)

    hist_k = pl.kernel(
        functools.partial(_hist_body, n_per_w),
        out_type=jax.ShapeDtypeStruct((NW * _HSLOTS,), jnp.float32),
        mesh=mesh,
        compiler_params=cparams,
        scratch_types=[
            pltpu.VMEM((n_per_w,), jnp.float32),
            pltpu.VMEM((n_per_w,), jnp.float32),
            pltpu.VMEM((L * _HSLOTS,), jnp.float32),
            pltpu.VMEM((_HSLOTS,), jnp.float32),
            pltpu.SemaphoreType.DMA,
        ],
    )
    hp = hist_k(srcf, tgtf)

    apply_k = pl.kernel(
        functools.partial(_apply_body, n_per_w, npix_c),
        out_type=jax.ShapeDtypeStruct((npix,), jnp.float32),
        mesh=mesh,
        compiler_params=cparams,
        scratch_types=[
            pltpu.VMEM((n_per_w,), jnp.float32),
            pltpu.VMEM((n_per_w,), jnp.float32),
            pltpu.VMEM((NW * _HSLOTS,), jnp.float32),
            pltpu.VMEM((_HSLOTS,), jnp.float32),
            pltpu.VMEM((_HSLOTS,), jnp.float32),
            pltpu.VMEM((3 * NBINS,), jnp.float32),
            pltpu.VMEM((NBINS,), jnp.float32),
            pltpu.SemaphoreType.DMA,
        ],
    )
    return apply_k(srcf, hp, fc)


def kernel(src, tgt):
    h, w, c = src.shape
    fc = jnp.asarray(_FC_NP)
    outf = _run(src.reshape(-1), tgt.reshape(-1), fc)
    return outf.reshape(h, w, c)


# speedup vs baseline: 232.6160x; 4.3137x over previous
import functools

import jax
import jax.numpy as jnp
import numpy as np
from jax import lax
from jax.experimental import pallas as pl
from jax.experimental.pallas import tpu as pltpu
from jax.experimental.pallas import tpu_sc as plsc

NBINS = 256
NC, NS = 2, 16
NW = NC * NS
L = 16

_FC_NP = np.clip(np.arange(-1.0, 1.01, 1.0 / 127.0), -1.0, 1.0).astype(np.float32)

_HSLOTS = 2 * 3 * NBINS


def _worker_id():
    return lax.axis_index("s") * NC + lax.axis_index("c")


def _bin_index(x):
    d = (x + jnp.float32(1.0)) / jnp.float32(2.0) * jnp.float32(256.0)
    d = jnp.minimum(jnp.maximum(d, jnp.float32(0.0)), jnp.float32(255.0))
    return d.astype(jnp.int32)


def _hist_body(n_per_w, src_hbm, tgt_hbm, hp_hbm, sv, tv, hb, hc, sem):
    wid = _worker_id()
    base = wid * n_per_w
    c1 = pltpu.async_copy(src_hbm.at[pl.ds(base, n_per_w)], sv, sem)
    c2 = pltpu.async_copy(tgt_hbm.at[pl.ds(base, n_per_w)], tv, sem)
    c1.wait()
    c2.wait()

    lanes = lax.iota(jnp.int32, L)
    ones = jnp.full((L,), 1.0, jnp.float32)
    zeros = jnp.zeros((L,), jnp.float32)
    chshift = 18

    def zero_body(i, _):
        hb[pl.ds(i * L, L)] = zeros
        return 0

    lax.fori_loop(0, L * _HSLOTS // L, zero_body, 0)

    nvec = n_per_w // L

    def px_body(i, _):
        off = i * L
        ch = jnp.right_shift(base + off + lanes, chshift)
        lane_slot = lanes * _HSLOTS + ch * NBINS
        x = sv[pl.ds(off, L)]
        plsc.addupdate_scatter(hb, [lane_slot + _bin_index(x)], ones)
        y = tv[pl.ds(off, L)]
        plsc.addupdate_scatter(hb, [lane_slot + (3 * NBINS) + _bin_index(y)], ones)
        return 0

    lax.fori_loop(0, nvec, px_body, 0)

    def red_body(s, _):
        acc = hb[pl.ds(s * L, L)]
        for l in range(1, L):
            acc = acc + hb[pl.ds(l * _HSLOTS + s * L, L)]
        hc[pl.ds(s * L, L)] = acc
        return 0

    lax.fori_loop(0, _HSLOTS // L, red_body, 0)

    pltpu.sync_copy(hc, hp_hbm.at[pl.ds(wid * _HSLOTS, _HSLOTS)])


def _lower_bound(a_ref, a_off, x):
    lb = jnp.zeros((L,), jnp.int32)
    step = NBINS
    while step >= 1:
        probe = lb + (step - 1)
        inb = probe < NBINS
        pv = plsc.load_gather(a_ref, [a_off + jnp.minimum(probe, NBINS - 1)])
        take = jnp.logical_and(inb, pv < x)
        lb = jnp.where(take, lb + step, lb)
        step //= 2
    return lb


def _apply_body(n_per_w, npix_c, src_hbm, hp_hbm, fc_hbm, out_hbm,
                sv, ov, hg, gh, cdfv, pxv, fcv, sem):
    wid = _worker_id()
    base = wid * n_per_w
    c1 = pltpu.async_copy(src_hbm.at[pl.ds(base, n_per_w)], sv, sem)
    c2 = pltpu.async_copy(hp_hbm, hg, sem)
    c3 = pltpu.async_copy(fc_hbm, fcv, sem)
    c1.wait()
    c2.wait()
    c3.wait()

    lanes = lax.iota(jnp.int32, L)
    f0 = jnp.float32(0.0)
    scale = jnp.float32(2.0)
    denom = jnp.float32(npix_c - 1)

    def red_body(s, _):
        acc = hg[pl.ds(s * L, L)]
        for w in range(1, NW):
            acc = acc + hg[pl.ds(w * _HSLOTS + s * L, L)]
        gh[pl.ds(s * L, L)] = acc
        return 0

    lax.fori_loop(0, _HSLOTS // L, red_body, 0)

    for slot in range(6):
        off = slot * NBINS
        carry = f0
        cdf0 = f0
        for k in range(NBINS // L):
            v = gh[pl.ds(off + k * L, L)]
            cs = plsc.cumsum(v) + carry
            if k == 0:
                cdf0 = jnp.sum(jnp.where(lanes == 0, cs, f0))
            carry = carry + jnp.sum(v)
            cdfv[pl.ds(off + k * L, L)] = (cs - cdf0) * scale / denom - jnp.float32(1.0)

    for ch in range(3):
        a_off = (3 + ch) * NBINS

        def q_body(q, _, a_off=a_off, ch=ch):
            x = cdfv[pl.ds(ch * NBINS + q * L, L)]
            lb = _lower_bound(cdfv, a_off, x)
            lbc = jnp.minimum(jnp.maximum(lb, 1), NBINS - 1)
            v1 = plsc.load_gather(cdfv, [a_off + lbc - 1])
            v2 = plsc.load_gather(cdfv, [a_off + lbc])
            below = jnp.abs(v1 - x) <= jnp.abs(v2 - x)
            fo1 = _lower_bound(cdfv, a_off, v1)
            ind1 = jnp.where(below, fo1, lbc)
            ind1 = jnp.where(lb == 0, 0, ind1)
            ind0 = jnp.maximum(ind1 - 1, 0)
            a0 = plsc.load_gather(cdfv, [a_off + ind0])
            a1 = plsc.load_gather(cdfv, [a_off + ind1])
            y0 = plsc.load_gather(fcv, [ind0])
            y1 = plsc.load_gather(fcv, [ind1])
            inner = y0 + (y1 - y0) * (x - a0) / (a1 - a0)
            atop = plsc.load_gather(cdfv, [jnp.full((L,), a_off + NBINS - 1, jnp.int32)])
            res = jnp.where(x <= jnp.float32(-1.0), jnp.float32(-1.0),
                            jnp.where(x >= atop, jnp.float32(1.0), inner))
            pxv[pl.ds(ch * NBINS + q * L, L)] = res
            return 0

        lax.fori_loop(0, NBINS // L, q_body, 0)

    inf = jnp.float32(np.inf)

    def px_body(i, _):
        off = i * L
        x = sv[pl.ds(off, L)]
        ch = jnp.right_shift(base + off + lanes, 18)
        choff = ch * NBINS
        t = (x + jnp.float32(1.0)) * jnp.float32(127.0)
        t = jnp.minimum(jnp.maximum(t, f0), jnp.float32(255.0))
        k0 = jnp.minimum(jnp.maximum(t.astype(jnp.int32) - 1, 0), NBINS - 4)
        bd = jnp.full((L,), inf)
        bi = jnp.zeros((L,), jnp.int32)
        fc1 = jnp.zeros((L,), jnp.float32)
        for o in range(4):
            ki = k0 + o
            fck = plsc.load_gather(fcv, [ki])
            d = jnp.abs(fck - x)
            upd = d < bd
            bd = jnp.where(upd, d, bd)
            bi = jnp.where(upd, ki, bi)
            fc1 = jnp.where(upd, fck, fc1)
        ind0 = jnp.maximum(bi - 1, 0)
        fc0 = plsc.load_gather(fcv, [ind0])
        p0 = plsc.load_gather(pxv, [choff + ind0])
        p1 = plsc.load_gather(pxv, [choff + bi])
        inner = p0 + (p1 - p0) * (x - fc0) / (fc1 - fc0)
        plo = plsc.load_gather(pxv, [choff])
        phi = plsc.load_gather(pxv, [choff + NBINS - 1])
        res = jnp.where(x <= jnp.float32(-1.0), plo,
                        jnp.where(x >= jnp.float32(1.0), phi, inner))
        ov[pl.ds(off, L)] = res
        return 0

    lax.fori_loop(0, n_per_w // L, px_body, 0)

    pltpu.sync_copy(ov, out_hbm.at[pl.ds(base, n_per_w)])


@jax.jit
def _run(srcf, tgtf, fc):
    npix = srcf.shape[0]
    npix_c = npix // 3
    n_per_w = npix // NW
    mesh = plsc.VectorSubcoreMesh(
        core_axis_name="c", subcore_axis_name="s", num_cores=NC, num_subcores=NS)
    cparams = pltpu.CompilerParams(needs_layout_passes=False)

    hist_k = pl.kernel(
        functools.partial(_hist_body, n_per_w),
        out_type=jax.ShapeDtypeStruct((NW * _HSLOTS,), jnp.float32),
        mesh=mesh,
        compiler_params=cparams,
        scratch_types=[
            pltpu.VMEM((n_per_w,), jnp.float32),
            pltpu.VMEM((n_per_w,), jnp.float32),
            pltpu.VMEM((L * _HSLOTS,), jnp.float32),
            pltpu.VMEM((_HSLOTS,), jnp.float32),
            pltpu.SemaphoreType.DMA,
        ],
    )
    hp = hist_k(srcf, tgtf)

    apply_k = pl.kernel(
        functools.partial(_apply_body, n_per_w, npix_c),
        out_type=jax.ShapeDtypeStruct((npix,), jnp.float32),
        mesh=mesh,
        compiler_params=cparams,
        scratch_types=[
            pltpu.VMEM((n_per_w,), jnp.float32),
            pltpu.VMEM((n_per_w,), jnp.float32),
            pltpu.VMEM((NW * _HSLOTS,), jnp.float32),
            pltpu.VMEM((_HSLOTS,), jnp.float32),
            pltpu.VMEM((_HSLOTS,), jnp.float32),
            pltpu.VMEM((3 * NBINS,), jnp.float32),
            pltpu.VMEM((NBINS,), jnp.float32),
            pltpu.SemaphoreType.DMA,
        ],
    )
    return apply_k(srcf, hp, fc)


def kernel(src, tgt):
    h, w, c = src.shape
    fc = jnp.asarray(_FC_NP)
    srcf = jnp.transpose(src, (2, 0, 1)).reshape(-1)
    tgtf = jnp.transpose(tgt, (2, 0, 1)).reshape(-1)
    outf = _run(srcf, tgtf, fc)
    return jnp.transpose(outf.reshape(c, h, w), (1, 2, 0))


# speedup vs baseline: 419.2740x; 1.8024x over previous
import functools

import jax
import jax.numpy as jnp
import numpy as np
from jax import lax
from jax.experimental import pallas as pl
from jax.experimental.pallas import tpu as pltpu
from jax.experimental.pallas import tpu_sc as plsc

NBINS = 256
NC, NS = 2, 16
NW = NC * NS
L = 16

_FC_NP = np.clip(np.arange(-1.0, 1.01, 1.0 / 127.0), -1.0, 1.0).astype(np.float32)

_HSLOTS = 2 * 3 * NBINS


def _worker_id():
    return lax.axis_index("s") * NC + lax.axis_index("c")


def _bin_index(x):
    d = (x + jnp.float32(1.0)) / jnp.float32(2.0) * jnp.float32(256.0)
    d = jnp.minimum(jnp.maximum(d, jnp.float32(0.0)), jnp.float32(255.0))
    return d.astype(jnp.int32)


def _seg_bounds(base, n_per_w, npix_c, c):
    lo = jnp.clip(c * npix_c - base, 0, n_per_w)
    hi = jnp.clip((c + 1) * npix_c - base, 0, n_per_w)
    return lo, hi


def _hist_body(n_per_w, npix_c, src_hbm, tgt_hbm, hp_hbm, sv, tv, hb, hc, sem):
    wid = _worker_id()
    base = wid * n_per_w
    c1 = pltpu.async_copy(src_hbm.at[pl.ds(base, n_per_w)], sv, sem)
    c2 = pltpu.async_copy(tgt_hbm.at[pl.ds(base, n_per_w)], tv, sem)

    lanes = lax.iota(jnp.int32, L)
    ones = jnp.full((L,), 1.0, jnp.float32)
    zeros = jnp.zeros((L,), jnp.float32)

    @plsc.parallel_loop(0, L * _HSLOTS, step=L, unroll=8)
    def _(i):
        hb[pl.ds(i, L)] = zeros

    c1.wait()
    c2.wait()

    for img, pv in ((0, sv), (1, tv)):
        for ch in range(3):
            lo, hi = _seg_bounds(base, n_per_w, npix_c, ch)
            pre = lanes * _HSLOTS + ((img * 3 + ch) * NBINS)

            @plsc.parallel_loop(lo, hi, step=L, unroll=8)
            def _(i, pv=pv, pre=pre):
                plsc.addupdate_scatter(hb, [pre + _bin_index(pv[pl.ds(i, L)])], ones)

    @plsc.parallel_loop(0, _HSLOTS, step=L, unroll=2)
    def _(s):
        acc = hb[pl.ds(s, L)]
        for l in range(1, L):
            acc = acc + hb[pl.ds(l * _HSLOTS + s, L)]
        hc[pl.ds(s, L)] = acc

    pltpu.sync_copy(hc, hp_hbm.at[pl.ds(wid * _HSLOTS, _HSLOTS)])


def _lower_bound(a_ref, a_off, x):
    lb = jnp.zeros((L,), jnp.int32)
    step = NBINS
    while step >= 1:
        probe = lb + (step - 1)
        inb = probe < NBINS
        pv = plsc.load_gather(a_ref, [a_off + jnp.minimum(probe, NBINS - 1)])
        take = jnp.logical_and(inb, pv < x)
        lb = jnp.where(take, lb + step, lb)
        step //= 2
    return lb


def _apply_body(n_per_w, npix_c, src_hbm, hp_hbm, fc_hbm, out_hbm,
                sv, ov, hg, gh, cdfv, pxv, fcv, sem):
    wid = _worker_id()
    base = wid * n_per_w
    c1 = pltpu.async_copy(src_hbm.at[pl.ds(base, n_per_w)], sv, sem)
    c2 = pltpu.async_copy(hp_hbm, hg, sem)
    c3 = pltpu.async_copy(fc_hbm, fcv, sem)
    c2.wait()
    c3.wait()

    lanes = lax.iota(jnp.int32, L)
    f0 = jnp.float32(0.0)
    scale = jnp.float32(2.0)
    denom = jnp.float32(npix_c - 1)

    @plsc.parallel_loop(0, _HSLOTS, step=L, unroll=2)
    def _(s):
        acc = hg[pl.ds(s, L)]
        for w in range(1, NW):
            acc = acc + hg[pl.ds(w * _HSLOTS + s, L)]
        gh[pl.ds(s, L)] = acc

    for slot in range(6):
        off = slot * NBINS
        carry = f0
        cdf0 = f0
        for k in range(NBINS // L):
            v = gh[pl.ds(off + k * L, L)]
            cs = plsc.cumsum(v) + carry
            if k == 0:
                cdf0 = jnp.sum(jnp.where(lanes == 0, cs, f0))
            carry = carry + jnp.sum(v)
            cdfv[pl.ds(off + k * L, L)] = (cs - cdf0) * scale / denom - jnp.float32(1.0)

    for ch in range(3):
        a_off = (3 + ch) * NBINS

        @plsc.parallel_loop(0, NBINS, step=L, unroll=4)
        def _(q, a_off=a_off, ch=ch):
            x = cdfv[pl.ds(ch * NBINS + q, L)]
            lb = _lower_bound(cdfv, a_off, x)
            lbc = jnp.minimum(jnp.maximum(lb, 1), NBINS - 1)
            v1 = plsc.load_gather(cdfv, [a_off + lbc - 1])
            v2 = plsc.load_gather(cdfv, [a_off + lbc])
            below = jnp.abs(v1 - x) <= jnp.abs(v2 - x)
            fo1 = _lower_bound(cdfv, a_off, v1)
            ind1 = jnp.where(below, fo1, lbc)
            ind1 = jnp.where(lb == 0, 0, ind1)
            ind0 = jnp.maximum(ind1 - 1, 0)
            a0 = plsc.load_gather(cdfv, [a_off + ind0])
            a1 = plsc.load_gather(cdfv, [a_off + ind1])
            y0 = plsc.load_gather(fcv, [ind0])
            y1 = plsc.load_gather(fcv, [ind1])
            inner = y0 + (y1 - y0) * (x - a0) / (a1 - a0)
            atop = plsc.load_gather(cdfv, [jnp.full((L,), a_off + NBINS - 1, jnp.int32)])
            res = jnp.where(x <= jnp.float32(-1.0), jnp.float32(-1.0),
                            jnp.where(x >= atop, jnp.float32(1.0), inner))
            pxv[pl.ds(ch * NBINS + q, L)] = res

    c1.wait()

    zero_i = jnp.zeros((L,), jnp.int32)
    for ch in range(3):
        lo, hi = _seg_bounds(base, n_per_w, npix_c, ch)
        choff = ch * NBINS
        plo = plsc.load_gather(pxv, [zero_i + choff])
        phi = plsc.load_gather(pxv, [zero_i + (choff + NBINS - 1)])

        @plsc.parallel_loop(lo, hi, step=L, unroll=8)
        def _(i, choff=choff, plo=plo, phi=phi):
            x = sv[pl.ds(i, L)]
            t = (x + jnp.float32(1.0)) * jnp.float32(127.0) + jnp.float32(0.5)
            t = jnp.minimum(jnp.maximum(t, f0), jnp.float32(255.0))
            k0 = jnp.minimum(jnp.maximum(t.astype(jnp.int32), 1), NBINS - 2) - 1
            bd = jnp.full((L,), jnp.float32(np.inf))
            bi = zero_i
            fc1 = jnp.zeros((L,), jnp.float32)
            for o in range(3):
                ki = k0 + o
                fck = plsc.load_gather(fcv, [ki])
                d = jnp.abs(fck - x)
                upd = d < bd
                bd = jnp.where(upd, d, bd)
                bi = jnp.where(upd, ki, bi)
                fc1 = jnp.where(upd, fck, fc1)
            ind0 = jnp.maximum(bi - 1, 0)
            fc0 = plsc.load_gather(fcv, [ind0])
            p0 = plsc.load_gather(pxv, [choff + ind0])
            p1 = plsc.load_gather(pxv, [choff + bi])
            inner = p0 + (p1 - p0) * (x - fc0) / (fc1 - fc0)
            res = jnp.where(x <= jnp.float32(-1.0), plo,
                            jnp.where(x >= jnp.float32(1.0), phi, inner))
            ov[pl.ds(i, L)] = res

    pltpu.sync_copy(ov, out_hbm.at[pl.ds(base, n_per_w)])


@jax.jit
def _run(srcf, tgtf, fc):
    npix = srcf.shape[0]
    npix_c = npix // 3
    n_per_w = npix // NW
    mesh = plsc.VectorSubcoreMesh(
        core_axis_name="c", subcore_axis_name="s", num_cores=NC, num_subcores=NS)
    cparams = pltpu.CompilerParams(needs_layout_passes=False)

    hist_k = pl.kernel(
        functools.partial(_hist_body, n_per_w, npix_c),
        out_type=jax.ShapeDtypeStruct((NW * _HSLOTS,), jnp.float32),
        mesh=mesh,
        compiler_params=cparams,
        scratch_types=[
            pltpu.VMEM((n_per_w,), jnp.float32),
            pltpu.VMEM((n_per_w,), jnp.float32),
            pltpu.VMEM((L * _HSLOTS,), jnp.float32),
            pltpu.VMEM((_HSLOTS,), jnp.float32),
            pltpu.SemaphoreType.DMA,
        ],
    )
    hp = hist_k(srcf, tgtf)

    apply_k = pl.kernel(
        functools.partial(_apply_body, n_per_w, npix_c),
        out_type=jax.ShapeDtypeStruct((npix,), jnp.float32),
        mesh=mesh,
        compiler_params=cparams,
        scratch_types=[
            pltpu.VMEM((n_per_w,), jnp.float32),
            pltpu.VMEM((n_per_w,), jnp.float32),
            pltpu.VMEM((NW * _HSLOTS,), jnp.float32),
            pltpu.VMEM((_HSLOTS,), jnp.float32),
            pltpu.VMEM((_HSLOTS,), jnp.float32),
            pltpu.VMEM((3 * NBINS,), jnp.float32),
            pltpu.VMEM((NBINS,), jnp.float32),
            pltpu.SemaphoreType.DMA,
        ],
    )
    return apply_k(srcf, hp, fc)


def kernel(src, tgt):
    h, w, c = src.shape
    fc = jnp.asarray(_FC_NP)
    srcf = jnp.transpose(src, (2, 0, 1)).reshape(-1)
    tgtf = jnp.transpose(tgt, (2, 0, 1)).reshape(-1)
    outf = _run(srcf, tgtf, fc)
    return jnp.transpose(outf.reshape(c, h, w), (1, 2, 0))


# speedup vs baseline: 432.1568x; 1.0307x over previous
import functools

import jax
import jax.numpy as jnp
import numpy as np
from jax import lax
from jax.experimental import pallas as pl
from jax.experimental.pallas import tpu as pltpu
from jax.experimental.pallas import tpu_sc as plsc

NBINS = 256
NC, NS = 2, 16
NW = NC * NS
L = 16

_FC_NP = np.clip(np.arange(-1.0, 1.01, 1.0 / 127.0), -1.0, 1.0).astype(np.float32)

_HSLOTS = 2 * 3 * NBINS


def _worker_id():
    return lax.axis_index("s") * NC + lax.axis_index("c")


def _bin_index(x):
    d = (x + jnp.float32(1.0)) / jnp.float32(2.0) * jnp.float32(256.0)
    d = jnp.minimum(jnp.maximum(d, jnp.float32(0.0)), jnp.float32(255.0))
    return d.astype(jnp.int32)


def _seg_bounds(base, n_per_w, npix_c, c):
    lo = jnp.clip(c * npix_c - base, 0, n_per_w)
    hi = jnp.clip((c + 1) * npix_c - base, 0, n_per_w)
    return lo, hi


def _hist_body(n_per_w, npix_c, src_hbm, tgt_hbm, hp_hbm,
               sv, tv, hb, hc, hall, shist, sem):
    sub = lax.axis_index("s")
    core = lax.axis_index("c")
    wid = sub * NC + core
    base = wid * n_per_w
    c1 = pltpu.async_copy(src_hbm.at[pl.ds(base, n_per_w)], sv, sem)
    c2 = pltpu.async_copy(tgt_hbm.at[pl.ds(base, n_per_w)], tv, sem)

    lanes = lax.iota(jnp.int32, L)
    ones = jnp.full((L,), 1.0, jnp.float32)
    zeros = jnp.zeros((L,), jnp.float32)

    @plsc.parallel_loop(0, L * _HSLOTS, step=L, unroll=8)
    def _(i):
        hb[pl.ds(i, L)] = zeros

    c1.wait()
    for img, pv in ((0, sv), (1, tv)):
        for ch in range(3):
            lo, hi = _seg_bounds(base, n_per_w, npix_c, ch)
            pre = lanes * _HSLOTS + ((img * 3 + ch) * NBINS)

            @plsc.parallel_loop(lo, hi, step=L, unroll=8)
            def _(i, pv=pv, pre=pre):
                plsc.addupdate_scatter(hb, [pre + _bin_index(pv[pl.ds(i, L)])], ones)

        if img == 0:
            c2.wait()

    @plsc.parallel_loop(0, _HSLOTS, step=L, unroll=2)
    def _(s):
        acc = hb[pl.ds(s, L)]
        for l in range(1, L):
            acc = acc + hb[pl.ds(l * _HSLOTS + s, L)]
        hc[pl.ds(s, L)] = acc

    pltpu.sync_copy(hc, shist.at[pl.ds(sub * _HSLOTS, _HSLOTS)])
    plsc.subcore_barrier()

    @pl.when(sub == 0)
    def _():
        pltpu.sync_copy(shist, hall)

        @plsc.parallel_loop(0, _HSLOTS, step=L, unroll=4)
        def _(s):
            acc = hall[pl.ds(s, L)]
            for l in range(1, NS):
                acc = acc + hall[pl.ds(l * _HSLOTS + s, L)]
            hc[pl.ds(s, L)] = acc

        pltpu.sync_copy(hc, hp_hbm.at[pl.ds(core * _HSLOTS, _HSLOTS)])


def _lower_bound(a_ref, a_off, x):
    lb = jnp.zeros((L,), jnp.int32)
    step = NBINS
    while step >= 1:
        probe = lb + (step - 1)
        inb = probe < NBINS
        pv = plsc.load_gather(a_ref, [a_off + jnp.minimum(probe, NBINS - 1)])
        take = jnp.logical_and(inb, pv < x)
        lb = jnp.where(take, lb + step, lb)
        step //= 2
    return lb


def _apply_body(n_per_w, npix_c, src_hbm, hp_hbm, fc_hbm, out_hbm,
                sv, ov, hg, gh, cdfv, pxv, fcv, sem):
    wid = _worker_id()
    base = wid * n_per_w
    c1 = pltpu.async_copy(src_hbm.at[pl.ds(base, n_per_w)], sv, sem)
    c2 = pltpu.async_copy(hp_hbm, hg, sem)
    c3 = pltpu.async_copy(fc_hbm, fcv, sem)
    c2.wait()
    c3.wait()

    lanes = lax.iota(jnp.int32, L)
    f0 = jnp.float32(0.0)
    scale = jnp.float32(2.0)
    denom = jnp.float32(npix_c - 1)

    @plsc.parallel_loop(0, _HSLOTS, step=L, unroll=4)
    def _(s):
        gh[pl.ds(s, L)] = hg[pl.ds(s, L)] + hg[pl.ds(_HSLOTS + s, L)]

    for slot in range(6):
        off = slot * NBINS
        carry = f0
        cdf0 = f0
        for k in range(NBINS // L):
            v = gh[pl.ds(off + k * L, L)]
            cs = plsc.cumsum(v) + carry
            if k == 0:
                cdf0 = jnp.sum(jnp.where(lanes == 0, cs, f0))
            carry = carry + jnp.sum(v)
            cdfv[pl.ds(off + k * L, L)] = (cs - cdf0) * scale / denom - jnp.float32(1.0)

    for ch in range(3):
        a_off = (3 + ch) * NBINS

        @plsc.parallel_loop(0, NBINS, step=L, unroll=4)
        def _(q, a_off=a_off, ch=ch):
            x = cdfv[pl.ds(ch * NBINS + q, L)]
            lb = _lower_bound(cdfv, a_off, x)
            lbc = jnp.minimum(jnp.maximum(lb, 1), NBINS - 1)
            v1 = plsc.load_gather(cdfv, [a_off + lbc - 1])
            v2 = plsc.load_gather(cdfv, [a_off + lbc])
            below = jnp.abs(v1 - x) <= jnp.abs(v2 - x)
            fo1 = _lower_bound(cdfv, a_off, v1)
            ind1 = jnp.where(below, fo1, lbc)
            ind1 = jnp.where(lb == 0, 0, ind1)
            ind0 = jnp.maximum(ind1 - 1, 0)
            a0 = plsc.load_gather(cdfv, [a_off + ind0])
            a1 = plsc.load_gather(cdfv, [a_off + ind1])
            y0 = plsc.load_gather(fcv, [ind0])
            y1 = plsc.load_gather(fcv, [ind1])
            inner = y0 + (y1 - y0) * (x - a0) / (a1 - a0)
            atop = plsc.load_gather(cdfv, [jnp.full((L,), a_off + NBINS - 1, jnp.int32)])
            res = jnp.where(x <= jnp.float32(-1.0), jnp.float32(-1.0),
                            jnp.where(x >= atop, jnp.float32(1.0), inner))
            pxv[pl.ds(ch * NBINS + q, L)] = res

    c1.wait()

    zero_i = jnp.zeros((L,), jnp.int32)
    for ch in range(3):
        lo, hi = _seg_bounds(base, n_per_w, npix_c, ch)
        choff = ch * NBINS
        plo = plsc.load_gather(pxv, [zero_i + choff])
        phi = plsc.load_gather(pxv, [zero_i + (choff + NBINS - 1)])

        @plsc.parallel_loop(lo, hi, step=L, unroll=8)
        def _(i, choff=choff, plo=plo, phi=phi):
            x = sv[pl.ds(i, L)]
            t = (x + jnp.float32(1.0)) * jnp.float32(127.0) + jnp.float32(0.5)
            t = jnp.minimum(jnp.maximum(t, f0), jnp.float32(255.0))
            k0 = jnp.minimum(jnp.maximum(t.astype(jnp.int32), 1), NBINS - 2) - 1
            bd = jnp.full((L,), jnp.float32(np.inf))
            bi = zero_i
            fc1 = jnp.zeros((L,), jnp.float32)
            for o in range(3):
                ki = k0 + o
                fck = plsc.load_gather(fcv, [ki])
                d = jnp.abs(fck - x)
                upd = d < bd
                bd = jnp.where(upd, d, bd)
                bi = jnp.where(upd, ki, bi)
                fc1 = jnp.where(upd, fck, fc1)
            ind0 = jnp.maximum(bi - 1, 0)
            fc0 = plsc.load_gather(fcv, [ind0])
            p0 = plsc.load_gather(pxv, [choff + ind0])
            p1 = plsc.load_gather(pxv, [choff + bi])
            inner = p0 + (p1 - p0) * (x - fc0) / (fc1 - fc0)
            res = jnp.where(x <= jnp.float32(-1.0), plo,
                            jnp.where(x >= jnp.float32(1.0), phi, inner))
            ov[pl.ds(i, L)] = res

    pltpu.sync_copy(ov, out_hbm.at[pl.ds(base, n_per_w)])


@jax.jit
def _run(srcf, tgtf, fc):
    npix = srcf.shape[0]
    npix_c = npix // 3
    n_per_w = npix // NW
    mesh = plsc.VectorSubcoreMesh(
        core_axis_name="c", subcore_axis_name="s", num_cores=NC, num_subcores=NS)
    cparams = pltpu.CompilerParams(needs_layout_passes=False)

    hist_k = pl.kernel(
        functools.partial(_hist_body, n_per_w, npix_c),
        out_type=jax.ShapeDtypeStruct((NC * _HSLOTS,), jnp.float32),
        mesh=mesh,
        compiler_params=cparams,
        scratch_types=[
            pltpu.VMEM((n_per_w,), jnp.float32),
            pltpu.VMEM((n_per_w,), jnp.float32),
            pltpu.VMEM((L * _HSLOTS,), jnp.float32),
            pltpu.VMEM((_HSLOTS,), jnp.float32),
            pltpu.VMEM((NS * _HSLOTS,), jnp.float32),
            pltpu.VMEM_SHARED((NS * _HSLOTS,), jnp.float32),
            pltpu.SemaphoreType.DMA,
        ],
    )
    hp = hist_k(srcf, tgtf)

    apply_k = pl.kernel(
        functools.partial(_apply_body, n_per_w, npix_c),
        out_type=jax.ShapeDtypeStruct((npix,), jnp.float32),
        mesh=mesh,
        compiler_params=cparams,
        scratch_types=[
            pltpu.VMEM((n_per_w,), jnp.float32),
            pltpu.VMEM((n_per_w,), jnp.float32),
            pltpu.VMEM((NC * _HSLOTS,), jnp.float32),
            pltpu.VMEM((_HSLOTS,), jnp.float32),
            pltpu.VMEM((_HSLOTS,), jnp.float32),
            pltpu.VMEM((3 * NBINS,), jnp.float32),
            pltpu.VMEM((NBINS,), jnp.float32),
            pltpu.SemaphoreType.DMA,
        ],
    )
    return apply_k(srcf, hp, fc)


def kernel(src, tgt):
    h, w, c = src.shape
    fc = jnp.asarray(_FC_NP)
    srcf = jnp.transpose(src, (2, 0, 1)).reshape(-1)
    tgtf = jnp.transpose(tgt, (2, 0, 1)).reshape(-1)
    outf = _run(srcf, tgtf, fc)
    return jnp.transpose(outf.reshape(c, h, w), (1, 2, 0))


# speedup vs baseline: 433.2791x; 1.0026x over previous
import functools

import jax
import jax.numpy as jnp
import numpy as np
from jax import lax
from jax.experimental import pallas as pl
from jax.experimental.pallas import tpu as pltpu
from jax.experimental.pallas import tpu_sc as plsc

NBINS = 256
NC, NS = 2, 16
NW = NC * NS
L = 16

_FC_NP = np.clip(np.arange(-1.0, 1.01, 1.0 / 127.0), -1.0, 1.0).astype(np.float32)

_HSLOTS = 2 * 3 * NBINS


def _worker_id():
    return lax.axis_index("s") * NC + lax.axis_index("c")


def _bin_index(x):
    d = (x + jnp.float32(1.0)) / jnp.float32(2.0) * jnp.float32(256.0)
    d = jnp.minimum(jnp.maximum(d, jnp.float32(0.0)), jnp.float32(255.0))
    return d.astype(jnp.int32)


def _seg_bounds(base, n_per_w, npix_c, c):
    lo = jnp.clip(c * npix_c - base, 0, n_per_w)
    hi = jnp.clip((c + 1) * npix_c - base, 0, n_per_w)
    return lo, hi


def _hist_body(n_per_w, npix_c, src_hbm, tgt_hbm, hp_hbm,
               sv, tv, hb, hc, hall, shist, sem):
    sub = lax.axis_index("s")
    core = lax.axis_index("c")
    wid = sub * NC + core
    base = wid * n_per_w
    c1 = pltpu.async_copy(src_hbm.at[pl.ds(base, n_per_w)], sv, sem)
    c2 = pltpu.async_copy(tgt_hbm.at[pl.ds(base, n_per_w)], tv, sem)

    lanes = lax.iota(jnp.int32, L)
    ones = jnp.full((L,), 1.0, jnp.float32)
    zeros = jnp.zeros((L,), jnp.float32)

    @plsc.parallel_loop(0, L * _HSLOTS, step=L, unroll=8)
    def _(i):
        hb[pl.ds(i, L)] = zeros

    c1.wait()
    for img, pv in ((0, sv), (1, tv)):
        for ch in range(3):
            lo, hi = _seg_bounds(base, n_per_w, npix_c, ch)
            pre = lanes * _HSLOTS + ((img * 3 + ch) * NBINS)

            @plsc.parallel_loop(lo, hi, step=L, unroll=8)
            def _(i, pv=pv, pre=pre):
                plsc.addupdate_scatter(hb, [pre + _bin_index(pv[pl.ds(i, L)])], ones)

        if img == 0:
            c2.wait()

    @plsc.parallel_loop(0, _HSLOTS, step=L, unroll=2)
    def _(s):
        acc = hb[pl.ds(s, L)]
        for l in range(1, L):
            acc = acc + hb[pl.ds(l * _HSLOTS + s, L)]
        hc[pl.ds(s, L)] = acc

    pltpu.sync_copy(hc, shist.at[pl.ds(sub * _HSLOTS, _HSLOTS)])
    plsc.subcore_barrier()

    @pl.when(sub == 0)
    def _():
        pltpu.sync_copy(shist, hall)

        @plsc.parallel_loop(0, _HSLOTS, step=L, unroll=4)
        def _(s):
            acc = hall[pl.ds(s, L)]
            for l in range(1, NS):
                acc = acc + hall[pl.ds(l * _HSLOTS + s, L)]
            hc[pl.ds(s, L)] = acc

        pltpu.sync_copy(hc, hp_hbm.at[pl.ds(core * _HSLOTS, _HSLOTS)])


def _lower_bound(a_ref, a_off, x):
    lb = jnp.zeros((L,), jnp.int32)
    step = NBINS
    while step >= 1:
        probe = lb + (step - 1)
        inb = probe < NBINS
        pv = plsc.load_gather(a_ref, [a_off + jnp.minimum(probe, NBINS - 1)])
        take = jnp.logical_and(inb, pv < x)
        lb = jnp.where(take, lb + step, lb)
        step //= 2
    return lb


def _apply_body(n_per_w, npix_c, src_hbm, hp_hbm, fc_hbm, out_hbm,
                sv, ov, hg, gh, cdfv, pxv, fcv, sem):
    wid = _worker_id()
    base = wid * n_per_w
    c1 = pltpu.async_copy(src_hbm.at[pl.ds(base, n_per_w)], sv, sem)
    c2 = pltpu.async_copy(hp_hbm, hg, sem)
    c3 = pltpu.async_copy(fc_hbm, fcv, sem)
    c2.wait()
    c3.wait()

    lanes = lax.iota(jnp.int32, L)
    f0 = jnp.float32(0.0)
    scale = jnp.float32(2.0)
    denom = jnp.float32(npix_c - 1)

    with jax.named_scope("p_reduce"):
        @plsc.parallel_loop(0, _HSLOTS, step=L, unroll=4)
        def _(s):
            gh[pl.ds(s, L)] = hg[pl.ds(s, L)] + hg[pl.ds(_HSLOTS + s, L)]

    scope_cdf = jax.named_scope("p_cdf")
    scope_cdf.__enter__()
    for slot in range(6):
        off = slot * NBINS
        carry = f0
        cdf0 = f0
        for k in range(NBINS // L):
            v = gh[pl.ds(off + k * L, L)]
            cs = plsc.cumsum(v) + carry
            if k == 0:
                cdf0 = jnp.sum(jnp.where(lanes == 0, cs, f0))
            carry = carry + jnp.sum(v)
            cdfv[pl.ds(off + k * L, L)] = (cs - cdf0) * scale / denom - jnp.float32(1.0)

    scope_cdf.__exit__(None, None, None)

    scope_px = jax.named_scope("p_pxmap")
    scope_px.__enter__()
    for ch in range(3):
        a_off = (3 + ch) * NBINS

        @plsc.parallel_loop(0, NBINS, step=L, unroll=4)
        def _(q, a_off=a_off, ch=ch):
            x = cdfv[pl.ds(ch * NBINS + q, L)]
            lb = _lower_bound(cdfv, a_off, x)
            lbc = jnp.minimum(jnp.maximum(lb, 1), NBINS - 1)
            v1 = plsc.load_gather(cdfv, [a_off + lbc - 1])
            v2 = plsc.load_gather(cdfv, [a_off + lbc])
            below = jnp.abs(v1 - x) <= jnp.abs(v2 - x)
            fo1 = _lower_bound(cdfv, a_off, v1)
            ind1 = jnp.where(below, fo1, lbc)
            ind1 = jnp.where(lb == 0, 0, ind1)
            ind0 = jnp.maximum(ind1 - 1, 0)
            a0 = plsc.load_gather(cdfv, [a_off + ind0])
            a1 = plsc.load_gather(cdfv, [a_off + ind1])
            y0 = plsc.load_gather(fcv, [ind0])
            y1 = plsc.load_gather(fcv, [ind1])
            inner = y0 + (y1 - y0) * (x - a0) / (a1 - a0)
            atop = plsc.load_gather(cdfv, [jnp.full((L,), a_off + NBINS - 1, jnp.int32)])
            res = jnp.where(x <= jnp.float32(-1.0), jnp.float32(-1.0),
                            jnp.where(x >= atop, jnp.float32(1.0), inner))
            pxv[pl.ds(ch * NBINS + q, L)] = res

    scope_px.__exit__(None, None, None)
    c1.wait()

    scope_s4 = jax.named_scope("p_stage4")
    scope_s4.__enter__()
    zero_i = jnp.zeros((L,), jnp.int32)
    for ch in range(3):
        lo, hi = _seg_bounds(base, n_per_w, npix_c, ch)
        choff = ch * NBINS
        plo = plsc.load_gather(pxv, [zero_i + choff])
        phi = plsc.load_gather(pxv, [zero_i + (choff + NBINS - 1)])

        @plsc.parallel_loop(lo, hi, step=L, unroll=8)
        def _(i, choff=choff, plo=plo, phi=phi):
            x = sv[pl.ds(i, L)]
            t = (x + jnp.float32(1.0)) * jnp.float32(127.0) + jnp.float32(0.5)
            t = jnp.minimum(jnp.maximum(t, f0), jnp.float32(255.0))
            k0 = jnp.minimum(jnp.maximum(t.astype(jnp.int32), 1), NBINS - 2) - 1
            bd = jnp.full((L,), jnp.float32(np.inf))
            bi = zero_i
            fc1 = jnp.zeros((L,), jnp.float32)
            for o in range(3):
                ki = k0 + o
                fck = plsc.load_gather(fcv, [ki])
                d = jnp.abs(fck - x)
                upd = d < bd
                bd = jnp.where(upd, d, bd)
                bi = jnp.where(upd, ki, bi)
                fc1 = jnp.where(upd, fck, fc1)
            ind0 = jnp.maximum(bi - 1, 0)
            fc0 = plsc.load_gather(fcv, [ind0])
            p0 = plsc.load_gather(pxv, [choff + ind0])
            p1 = plsc.load_gather(pxv, [choff + bi])
            inner = p0 + (p1 - p0) * (x - fc0) / (fc1 - fc0)
            res = jnp.where(x <= jnp.float32(-1.0), plo,
                            jnp.where(x >= jnp.float32(1.0), phi, inner))
            ov[pl.ds(i, L)] = res

    scope_s4.__exit__(None, None, None)
    pltpu.sync_copy(ov, out_hbm.at[pl.ds(base, n_per_w)])


@jax.jit
def _run(srcf, tgtf, fc):
    npix = srcf.shape[0]
    npix_c = npix // 3
    n_per_w = npix // NW
    mesh = plsc.VectorSubcoreMesh(
        core_axis_name="c", subcore_axis_name="s", num_cores=NC, num_subcores=NS)
    cparams = pltpu.CompilerParams(needs_layout_passes=False)

    hist_k = pl.kernel(
        functools.partial(_hist_body, n_per_w, npix_c),
        out_type=jax.ShapeDtypeStruct((NC * _HSLOTS,), jnp.float32),
        mesh=mesh,
        compiler_params=cparams,
        scratch_types=[
            pltpu.VMEM((n_per_w,), jnp.float32),
            pltpu.VMEM((n_per_w,), jnp.float32),
            pltpu.VMEM((L * _HSLOTS,), jnp.float32),
            pltpu.VMEM((_HSLOTS,), jnp.float32),
            pltpu.VMEM((NS * _HSLOTS,), jnp.float32),
            pltpu.VMEM_SHARED((NS * _HSLOTS,), jnp.float32),
            pltpu.SemaphoreType.DMA,
        ],
    )
    hp = hist_k(srcf, tgtf)

    apply_k = pl.kernel(
        functools.partial(_apply_body, n_per_w, npix_c),
        out_type=jax.ShapeDtypeStruct((npix,), jnp.float32),
        mesh=mesh,
        compiler_params=cparams,
        scratch_types=[
            pltpu.VMEM((n_per_w,), jnp.float32),
            pltpu.VMEM((n_per_w,), jnp.float32),
            pltpu.VMEM((NC * _HSLOTS,), jnp.float32),
            pltpu.VMEM((_HSLOTS,), jnp.float32),
            pltpu.VMEM((_HSLOTS,), jnp.float32),
            pltpu.VMEM((3 * NBINS,), jnp.float32),
            pltpu.VMEM((NBINS,), jnp.float32),
            pltpu.SemaphoreType.DMA,
        ],
    )
    return apply_k(srcf, hp, fc)


def kernel(src, tgt):
    h, w, c = src.shape
    fc = jnp.asarray(_FC_NP)
    srcf = jnp.transpose(src, (2, 0, 1)).reshape(-1)
    tgtf = jnp.transpose(tgt, (2, 0, 1)).reshape(-1)
    outf = _run(srcf, tgtf, fc)
    return jnp.transpose(outf.reshape(c, h, w), (1, 2, 0))


# speedup vs baseline: 558.8722x; 1.2899x over previous
import functools

import jax
import jax.numpy as jnp
import numpy as np
from jax import lax
from jax.experimental import pallas as pl
from jax.experimental.pallas import tpu as pltpu
from jax.experimental.pallas import tpu_sc as plsc

NBINS = 256
NC, NS = 2, 16
NW = NC * NS
L = 16

_FC_NP = np.clip(np.arange(-1.0, 1.01, 1.0 / 127.0), -1.0, 1.0).astype(np.float32)

_HSLOTS = 2 * 3 * NBINS


def _worker_id():
    return lax.axis_index("s") * NC + lax.axis_index("c")


def _bin_index(x):
    d = (x + jnp.float32(1.0)) / jnp.float32(2.0) * jnp.float32(256.0)
    d = jnp.minimum(jnp.maximum(d, jnp.float32(0.0)), jnp.float32(255.0))
    return d.astype(jnp.int32)


def _seg_bounds(base, n_per_w, npix_c, c):
    lo = jnp.clip(c * npix_c - base, 0, n_per_w)
    hi = jnp.clip((c + 1) * npix_c - base, 0, n_per_w)
    return lo, hi


def _hist_body(n_per_w, npix_c, src_hbm, tgt_hbm, hp_hbm,
               sv, tv, hb, hc, hall, shist, sem):
    sub = lax.axis_index("s")
    core = lax.axis_index("c")
    wid = sub * NC + core
    base = wid * n_per_w
    c1 = pltpu.async_copy(src_hbm.at[pl.ds(base, n_per_w)], sv, sem)
    c2 = pltpu.async_copy(tgt_hbm.at[pl.ds(base, n_per_w)], tv, sem)

    lanes = lax.iota(jnp.int32, L)
    ones = jnp.full((L,), 1.0, jnp.float32)
    zeros = jnp.zeros((L,), jnp.float32)

    @plsc.parallel_loop(0, L * _HSLOTS, step=L, unroll=8)
    def _(i):
        hb[pl.ds(i, L)] = zeros

    c1.wait()
    for img, pv in ((0, sv), (1, tv)):
        for ch in range(3):
            lo, hi = _seg_bounds(base, n_per_w, npix_c, ch)
            pre = lanes * _HSLOTS + ((img * 3 + ch) * NBINS)

            @plsc.parallel_loop(lo, hi, step=L, unroll=8)
            def _(i, pv=pv, pre=pre):
                plsc.addupdate_scatter(hb, [pre + _bin_index(pv[pl.ds(i, L)])], ones)

        if img == 0:
            c2.wait()

    @plsc.parallel_loop(0, _HSLOTS, step=L, unroll=2)
    def _(s):
        acc = hb[pl.ds(s, L)]
        for l in range(1, L):
            acc = acc + hb[pl.ds(l * _HSLOTS + s, L)]
        hc[pl.ds(s, L)] = acc

    pltpu.sync_copy(hc, shist.at[pl.ds(sub * _HSLOTS, _HSLOTS)])
    plsc.subcore_barrier()

    @pl.when(sub == 0)
    def _():
        pltpu.sync_copy(shist, hall)

        @plsc.parallel_loop(0, _HSLOTS, step=L, unroll=4)
        def _(s):
            acc = hall[pl.ds(s, L)]
            for l in range(1, NS):
                acc = acc + hall[pl.ds(l * _HSLOTS + s, L)]
            hc[pl.ds(s, L)] = acc

        pltpu.sync_copy(hc, hp_hbm.at[pl.ds(core * _HSLOTS, _HSLOTS)])


def _lower_bound(a_ref, a_off, x):
    lb = jnp.zeros((L,), jnp.int32)
    step = NBINS
    while step >= 1:
        probe = lb + (step - 1)
        inb = probe < NBINS
        pv = plsc.load_gather(a_ref, [a_off + jnp.minimum(probe, NBINS - 1)])
        take = jnp.logical_and(inb, pv < x)
        lb = jnp.where(take, lb + step, lb)
        step //= 2
    return lb


def _apply_body(n_per_w, npix_c, src_hbm, hp_hbm, fc_hbm, out_hbm,
                sv, ov, hg, gh, cdfv, pxv, fcv, av, bv, sem):
    wid = _worker_id()
    base = wid * n_per_w
    c1 = pltpu.async_copy(src_hbm.at[pl.ds(base, n_per_w)], sv, sem)
    c2 = pltpu.async_copy(hp_hbm, hg, sem)
    c3 = pltpu.async_copy(fc_hbm, fcv, sem)
    c2.wait()
    c3.wait()

    lanes = lax.iota(jnp.int32, L)
    f0 = jnp.float32(0.0)
    scale = jnp.float32(2.0)
    denom = jnp.float32(npix_c - 1)

    with jax.named_scope("p_reduce"):
        @plsc.parallel_loop(0, _HSLOTS, step=L, unroll=4)
        def _(s):
            gh[pl.ds(s, L)] = hg[pl.ds(s, L)] + hg[pl.ds(_HSLOTS + s, L)]

    scope_cdf = jax.named_scope("p_cdf")
    scope_cdf.__enter__()
    for slot in range(6):
        off = slot * NBINS
        carry = f0
        cdf0 = f0
        for k in range(NBINS // L):
            v = gh[pl.ds(off + k * L, L)]
            cs = plsc.cumsum(v) + carry
            if k == 0:
                cdf0 = jnp.sum(jnp.where(lanes == 0, cs, f0))
            carry = carry + jnp.sum(v)
            cdfv[pl.ds(off + k * L, L)] = (cs - cdf0) * scale / denom - jnp.float32(1.0)

    scope_cdf.__exit__(None, None, None)

    scope_px = jax.named_scope("p_pxmap")
    scope_px.__enter__()
    for ch in range(3):
        a_off = (3 + ch) * NBINS

        @plsc.parallel_loop(0, NBINS, step=L, unroll=4)
        def _(q, a_off=a_off, ch=ch):
            x = cdfv[pl.ds(ch * NBINS + q, L)]
            lb = _lower_bound(cdfv, a_off, x)
            lbc = jnp.minimum(jnp.maximum(lb, 1), NBINS - 1)
            v1 = plsc.load_gather(cdfv, [a_off + lbc - 1])
            v2 = plsc.load_gather(cdfv, [a_off + lbc])
            below = jnp.abs(v1 - x) <= jnp.abs(v2 - x)
            fo1 = _lower_bound(cdfv, a_off, v1)
            ind1 = jnp.where(below, fo1, lbc)
            ind1 = jnp.where(lb == 0, 0, ind1)
            ind0 = jnp.maximum(ind1 - 1, 0)
            a0 = plsc.load_gather(cdfv, [a_off + ind0])
            a1 = plsc.load_gather(cdfv, [a_off + ind1])
            y0 = plsc.load_gather(fcv, [ind0])
            y1 = plsc.load_gather(fcv, [ind1])
            inner = y0 + (y1 - y0) * (x - a0) / (a1 - a0)
            atop = plsc.load_gather(cdfv, [jnp.full((L,), a_off + NBINS - 1, jnp.int32)])
            res = jnp.where(x <= jnp.float32(-1.0), jnp.float32(-1.0),
                            jnp.where(x >= atop, jnp.float32(1.0), inner))
            pxv[pl.ds(ch * NBINS + q, L)] = res

    scope_px.__exit__(None, None, None)

    with jax.named_scope("p_ab"):
        for ch in range(3):
            choff = ch * NBINS

            @plsc.parallel_loop(0, NBINS, step=L, unroll=4)
            def _(i, choff=choff):
                k = jnp.minimum(i + lanes, NBINS - 2)
                i0 = jnp.maximum(k - 1, 0)
                f1 = plsc.load_gather(fcv, [k])
                fv0 = plsc.load_gather(fcv, [i0])
                p1 = plsc.load_gather(pxv, [choff + k])
                p0 = plsc.load_gather(pxv, [choff + i0])
                b = (p1 - p0) / (f1 - fv0)
                av[pl.ds(choff + i, L)] = p0 - b * fv0
                bv[pl.ds(choff + i, L)] = b

    c1.wait()

    scope_s4 = jax.named_scope("p_stage4")
    scope_s4.__enter__()
    zero_i = jnp.zeros((L,), jnp.int32)
    for ch in range(3):
        lo, hi = _seg_bounds(base, n_per_w, npix_c, ch)
        choff = ch * NBINS
        plo = plsc.load_gather(pxv, [zero_i + choff])
        phi = plsc.load_gather(pxv, [zero_i + (choff + NBINS - 1)])

        @plsc.parallel_loop(lo, hi, step=L, unroll=8)
        def _(i, choff=choff, plo=plo, phi=phi):
            x = sv[pl.ds(i, L)]
            u = (x + jnp.float32(1.0)) * jnp.float32(127.0)
            u = jnp.minimum(jnp.maximum(u, f0), jnp.float32(254.0))
            g = jnp.minimum(u.astype(jnp.int32), NBINS - 3)
            d0 = jnp.abs(plsc.load_gather(fcv, [g]) - x)
            d1 = jnp.abs(plsc.load_gather(fcv, [g + 1]) - x)
            k = jnp.where(d0 <= d1, g, g + 1)
            a = plsc.load_gather(av, [choff + k])
            b = plsc.load_gather(bv, [choff + k])
            inner = a + b * x
            res = jnp.where(x <= jnp.float32(-1.0), plo,
                            jnp.where(x >= jnp.float32(1.0), phi, inner))
            ov[pl.ds(i, L)] = res

    scope_s4.__exit__(None, None, None)
    pltpu.sync_copy(ov, out_hbm.at[pl.ds(base, n_per_w)])


@jax.jit
def _run(srcf, tgtf, fc):
    npix = srcf.shape[0]
    npix_c = npix // 3
    n_per_w = npix // NW
    mesh = plsc.VectorSubcoreMesh(
        core_axis_name="c", subcore_axis_name="s", num_cores=NC, num_subcores=NS)
    cparams = pltpu.CompilerParams(needs_layout_passes=False)

    hist_k = pl.kernel(
        functools.partial(_hist_body, n_per_w, npix_c),
        out_type=jax.ShapeDtypeStruct((NC * _HSLOTS,), jnp.float32),
        mesh=mesh,
        compiler_params=cparams,
        scratch_types=[
            pltpu.VMEM((n_per_w,), jnp.float32),
            pltpu.VMEM((n_per_w,), jnp.float32),
            pltpu.VMEM((L * _HSLOTS,), jnp.float32),
            pltpu.VMEM((_HSLOTS,), jnp.float32),
            pltpu.VMEM((NS * _HSLOTS,), jnp.float32),
            pltpu.VMEM_SHARED((NS * _HSLOTS,), jnp.float32),
            pltpu.SemaphoreType.DMA,
        ],
    )
    hp = hist_k(srcf, tgtf)

    apply_k = pl.kernel(
        functools.partial(_apply_body, n_per_w, npix_c),
        out_type=jax.ShapeDtypeStruct((npix,), jnp.float32),
        mesh=mesh,
        compiler_params=cparams,
        scratch_types=[
            pltpu.VMEM((n_per_w,), jnp.float32),
            pltpu.VMEM((n_per_w,), jnp.float32),
            pltpu.VMEM((NC * _HSLOTS,), jnp.float32),
            pltpu.VMEM((_HSLOTS,), jnp.float32),
            pltpu.VMEM((_HSLOTS,), jnp.float32),
            pltpu.VMEM((3 * NBINS,), jnp.float32),
            pltpu.VMEM((NBINS,), jnp.float32),
            pltpu.VMEM((3 * NBINS,), jnp.float32),
            pltpu.VMEM((3 * NBINS,), jnp.float32),
            pltpu.SemaphoreType.DMA,
        ],
    )
    return apply_k(srcf, hp, fc)


def kernel(src, tgt):
    h, w, c = src.shape
    fc = jnp.asarray(_FC_NP)
    srcf = jnp.transpose(src, (2, 0, 1)).reshape(-1)
    tgtf = jnp.transpose(tgt, (2, 0, 1)).reshape(-1)
    outf = _run(srcf, tgtf, fc)
    return jnp.transpose(outf.reshape(c, h, w), (1, 2, 0))


# speedup vs baseline: 668.5610x; 1.1963x over previous
import functools

import jax
import jax.numpy as jnp
import numpy as np
from jax import lax
from jax.experimental import pallas as pl
from jax.experimental.pallas import tpu as pltpu
from jax.experimental.pallas import tpu_sc as plsc

NBINS = 256
NC, NS = 2, 16
NW = NC * NS
L = 16

_FC_NP = np.clip(np.arange(-1.0, 1.01, 1.0 / 127.0), -1.0, 1.0).astype(np.float32)

_HSLOTS = 2 * 3 * NBINS


def _worker_id():
    return lax.axis_index("s") * NC + lax.axis_index("c")


def _bin_index(x):
    d = (x + jnp.float32(1.0)) / jnp.float32(2.0) * jnp.float32(256.0)
    d = jnp.minimum(jnp.maximum(d, jnp.float32(0.0)), jnp.float32(255.0))
    return d.astype(jnp.int32)


def _hist_body(rows_w, src_hbm, tgt_hbm, hp_hbm,
               sv, tv, hb, hc, hall, shist, sem):
    sub = lax.axis_index("s")
    core = lax.axis_index("c")
    wid = sub * NC + core
    r0 = wid * rows_w
    ncol = src_hbm.shape[2]
    plane_px = rows_w * ncol
    copies = []
    for c in range(3):
        copies.append(pltpu.async_copy(src_hbm.at[c, pl.ds(r0, rows_w), :],
                                       sv.at[c], sem))
    for c in range(3):
        copies.append(pltpu.async_copy(tgt_hbm.at[c, pl.ds(r0, rows_w), :],
                                       tv.at[c], sem))

    lanes = lax.iota(jnp.int32, L)
    ones = jnp.full((L,), 1.0, jnp.float32)
    zeros = jnp.zeros((L,), jnp.float32)

    @plsc.parallel_loop(0, L * _HSLOTS, step=L, unroll=8)
    def _(i):
        hb[pl.ds(i, L)] = zeros

    for img, pv in ((0, sv), (1, tv)):
        for ch in range(3):
            copies[img * 3 + ch].wait()
            pre = lanes * _HSLOTS + ((img * 3 + ch) * NBINS)

            cshift = ncol.bit_length() - 1

            @plsc.parallel_loop(0, plane_px, step=L, unroll=8)
            def _(i, pv=pv, pre=pre, ch=ch, cshift=cshift):
                r = jnp.right_shift(i, cshift)
                col = jnp.bitwise_and(i, ncol - 1)
                plsc.addupdate_scatter(
                    hb, [pre + _bin_index(pv[ch, r, pl.ds(col, L)])], ones)

    @plsc.parallel_loop(0, _HSLOTS, step=L, unroll=2)
    def _(s):
        acc = hb[pl.ds(s, L)]
        for l in range(1, L):
            acc = acc + hb[pl.ds(l * _HSLOTS + s, L)]
        hc[pl.ds(s, L)] = acc

    pltpu.sync_copy(hc, shist.at[pl.ds(sub * _HSLOTS, _HSLOTS)])
    plsc.subcore_barrier()

    @pl.when(sub == 0)
    def _():
        pltpu.sync_copy(shist, hall)

        @plsc.parallel_loop(0, _HSLOTS, step=L, unroll=4)
        def _(s):
            acc = hall[pl.ds(s, L)]
            for l in range(1, NS):
                acc = acc + hall[pl.ds(l * _HSLOTS + s, L)]
            hc[pl.ds(s, L)] = acc

        pltpu.sync_copy(hc, hp_hbm.at[pl.ds(core * _HSLOTS, _HSLOTS)])


def _lower_bound(a_ref, a_off, x):
    lb = jnp.zeros((L,), jnp.int32)
    step = NBINS
    while step >= 1:
        probe = lb + (step - 1)
        inb = probe < NBINS
        pv = plsc.load_gather(a_ref, [a_off + jnp.minimum(probe, NBINS - 1)])
        take = jnp.logical_and(inb, pv < x)
        lb = jnp.where(take, lb + step, lb)
        step //= 2
    return lb


def _apply_body(rows_w, npix_c, src_hbm, hp_hbm, fc_hbm, out_hbm,
                sv, ov, hg, gh, cdfv, pxv, fcv, av, bv, sem):
    wid = _worker_id()
    r0 = wid * rows_w
    ncol = src_hbm.shape[2]
    plane_px = rows_w * ncol
    incopies = [pltpu.async_copy(src_hbm.at[c, pl.ds(r0, rows_w), :], sv.at[c], sem)
                for c in range(3)]
    c2 = pltpu.async_copy(hp_hbm, hg, sem)
    c3 = pltpu.async_copy(fc_hbm, fcv, sem)
    c2.wait()
    c3.wait()

    lanes = lax.iota(jnp.int32, L)
    f0 = jnp.float32(0.0)
    scale = jnp.float32(2.0)
    denom = jnp.float32(npix_c - 1)

    with jax.named_scope("p_reduce"):
        @plsc.parallel_loop(0, _HSLOTS, step=L, unroll=4)
        def _(s):
            gh[pl.ds(s, L)] = hg[pl.ds(s, L)] + hg[pl.ds(_HSLOTS + s, L)]

    scope_cdf = jax.named_scope("p_cdf")
    scope_cdf.__enter__()
    for slot in range(6):
        off = slot * NBINS
        carry = f0
        cdf0 = f0
        for k in range(NBINS // L):
            v = gh[pl.ds(off + k * L, L)]
            cs = plsc.cumsum(v) + carry
            if k == 0:
                cdf0 = jnp.sum(jnp.where(lanes == 0, cs, f0))
            carry = carry + jnp.sum(v)
            cdfv[pl.ds(off + k * L, L)] = (cs - cdf0) * scale / denom - jnp.float32(1.0)

    scope_cdf.__exit__(None, None, None)

    scope_px = jax.named_scope("p_pxmap")
    scope_px.__enter__()
    for ch in range(3):
        a_off = (3 + ch) * NBINS

        @plsc.parallel_loop(0, NBINS, step=L, unroll=4)
        def _(q, a_off=a_off, ch=ch):
            x = cdfv[pl.ds(ch * NBINS + q, L)]
            lb = _lower_bound(cdfv, a_off, x)
            lbc = jnp.minimum(jnp.maximum(lb, 1), NBINS - 1)
            v1 = plsc.load_gather(cdfv, [a_off + lbc - 1])
            v2 = plsc.load_gather(cdfv, [a_off + lbc])
            below = jnp.abs(v1 - x) <= jnp.abs(v2 - x)
            fo1 = _lower_bound(cdfv, a_off, v1)
            ind1 = jnp.where(below, fo1, lbc)
            ind1 = jnp.where(lb == 0, 0, ind1)
            ind0 = jnp.maximum(ind1 - 1, 0)
            a0 = plsc.load_gather(cdfv, [a_off + ind0])
            a1 = plsc.load_gather(cdfv, [a_off + ind1])
            y0 = plsc.load_gather(fcv, [ind0])
            y1 = plsc.load_gather(fcv, [ind1])
            inner = y0 + (y1 - y0) * (x - a0) / (a1 - a0)
            atop = plsc.load_gather(cdfv, [jnp.full((L,), a_off + NBINS - 1, jnp.int32)])
            res = jnp.where(x <= jnp.float32(-1.0), jnp.float32(-1.0),
                            jnp.where(x >= atop, jnp.float32(1.0), inner))
            pxv[pl.ds(ch * NBINS + q, L)] = res

    scope_px.__exit__(None, None, None)

    with jax.named_scope("p_ab"):
        for ch in range(3):
            choff = ch * NBINS

            @plsc.parallel_loop(0, NBINS, step=L, unroll=4)
            def _(i, choff=choff):
                k = jnp.minimum(i + lanes, NBINS - 2)
                i0 = jnp.maximum(k - 1, 0)
                f1 = plsc.load_gather(fcv, [k])
                fv0 = plsc.load_gather(fcv, [i0])
                p1 = plsc.load_gather(pxv, [choff + k])
                p0 = plsc.load_gather(pxv, [choff + i0])
                b = (p1 - p0) / (f1 - fv0)
                av[pl.ds(choff + i, L)] = p0 - b * fv0
                bv[pl.ds(choff + i, L)] = b

    scope_s4 = jax.named_scope("p_stage4")
    scope_s4.__enter__()
    zero_i = jnp.zeros((L,), jnp.int32)
    cshift = ncol.bit_length() - 1
    ocopies = []
    for ch in range(3):
        incopies[ch].wait()
        choff = ch * NBINS
        plo = plsc.load_gather(pxv, [zero_i + choff])
        phi = plsc.load_gather(pxv, [zero_i + (choff + NBINS - 1)])

        @plsc.parallel_loop(0, plane_px, step=L, unroll=8)
        def _(i, ch=ch, choff=choff, plo=plo, phi=phi):
            r = jnp.right_shift(i, cshift)
            col = jnp.bitwise_and(i, ncol - 1)
            x = sv[ch, r, pl.ds(col, L)]
            u = (x + jnp.float32(1.0)) * jnp.float32(127.0)
            u = jnp.minimum(jnp.maximum(u, f0), jnp.float32(254.0))
            g = jnp.minimum(u.astype(jnp.int32), NBINS - 3)
            d0 = jnp.abs(plsc.load_gather(fcv, [g]) - x)
            d1 = jnp.abs(plsc.load_gather(fcv, [g + 1]) - x)
            k = jnp.where(d0 <= d1, g, g + 1)
            a = plsc.load_gather(av, [choff + k])
            b = plsc.load_gather(bv, [choff + k])
            inner = a + b * x
            res = jnp.where(x <= jnp.float32(-1.0), plo,
                            jnp.where(x >= jnp.float32(1.0), phi, inner))
            ov[ch, r, pl.ds(col, L)] = res

        ocopies.append(pltpu.async_copy(
            ov.at[ch], out_hbm.at[ch, pl.ds(r0, rows_w), :], sem))

    scope_s4.__exit__(None, None, None)
    for oc in ocopies:
        oc.wait()


@jax.jit
def _run(srcT, tgtT, fc):
    _, h, w = srcT.shape
    npix_c = h * w
    rows_w = h // NW
    mesh = plsc.VectorSubcoreMesh(
        core_axis_name="c", subcore_axis_name="s", num_cores=NC, num_subcores=NS)
    cparams = pltpu.CompilerParams(needs_layout_passes=False)

    hist_k = pl.kernel(
        functools.partial(_hist_body, rows_w),
        out_type=jax.ShapeDtypeStruct((NC * _HSLOTS,), jnp.float32),
        mesh=mesh,
        compiler_params=cparams,
        scratch_types=[
            pltpu.VMEM((3, rows_w, w), jnp.float32),
            pltpu.VMEM((3, rows_w, w), jnp.float32),
            pltpu.VMEM((L * _HSLOTS,), jnp.float32),
            pltpu.VMEM((_HSLOTS,), jnp.float32),
            pltpu.VMEM((NS * _HSLOTS,), jnp.float32),
            pltpu.VMEM_SHARED((NS * _HSLOTS,), jnp.float32),
            pltpu.SemaphoreType.DMA,
        ],
    )
    hp = hist_k(srcT, tgtT)

    apply_k = pl.kernel(
        functools.partial(_apply_body, rows_w, npix_c),
        out_type=jax.ShapeDtypeStruct((3, h, w), jnp.float32),
        mesh=mesh,
        compiler_params=cparams,
        scratch_types=[
            pltpu.VMEM((3, rows_w, w), jnp.float32),
            pltpu.VMEM((3, rows_w, w), jnp.float32),
            pltpu.VMEM((NC * _HSLOTS,), jnp.float32),
            pltpu.VMEM((_HSLOTS,), jnp.float32),
            pltpu.VMEM((_HSLOTS,), jnp.float32),
            pltpu.VMEM((3 * NBINS,), jnp.float32),
            pltpu.VMEM((NBINS,), jnp.float32),
            pltpu.VMEM((3 * NBINS,), jnp.float32),
            pltpu.VMEM((3 * NBINS,), jnp.float32),
            pltpu.SemaphoreType.DMA,
        ],
    )
    return apply_k(srcT, hp, fc)


def kernel(src, tgt):
    fc = jnp.asarray(_FC_NP)
    outT = _run(jnp.transpose(src, (2, 0, 1)), jnp.transpose(tgt, (2, 0, 1)), fc)
    return jnp.transpose(outT, (1, 2, 0))
